# Initial kernel scaffold; baseline (speedup 1.0000x reference)
import functools

import jax
import jax.numpy as jnp
from jax import lax
from jax.experimental import pallas as pl
from jax.experimental.pallas import tpu as pltpu
from jax.experimental.pallas import tpu_sc as plsc

F32 = jnp.float32
I32 = jnp.int32

N_AST = 10000
N_TEST = 500
H = 128
E_AA = 320000
E_AT = 100000

NPA = 10240
NPT = 512
NW = 32
CH = 128
CH_AA = 79
CH_AT = 25
E_AA_P = NW * CH_AA * CH
E_AT_P = NW * CH_AT * CH
ROWS_A = NPA // 16
ROWS_T = NPT // 16

_MESH = plsc.VectorSubcoreMesh(core_axis_name="c", subcore_axis_name="s")


@functools.partial(
    pl.kernel,
    mesh=_MESH,
    out_type=(
        jax.ShapeDtypeStruct((NPA, 64), F32),
        jax.ShapeDtypeStruct((2, NPA, 16), F32),
        jax.ShapeDtypeStruct((2, NPT, 16), F32),
    ),
    scratch_types=[
        pltpu.VMEM((CH,), I32),
        pltpu.VMEM((CH, 16), F32),
        pltpu.VMEM((80,), I32),
        pltpu.VMEM((80, 64), F32),
        pltpu.VMEM_SHARED((NPA, 16), F32),
        pltpu.VMEM_SHARED((NPT, 16), F32),
        pltpu.SemaphoreType.DMA,
    ],
)
def _sc_init(lab_hbm, emb_tab_hbm, aa_dst_hbm, ta_dst_hbm, at_dst_hbm,
             z16_hbm, ones_hbm,
             emb_out, dega_out, degt_out,
             didx, onesv, lidx, erows, dega, degt, sem):
    c = lax.axis_index("c")
    s = lax.axis_index("s")
    w = c * 16 + s
    pltpu.sync_copy(z16_hbm, dega.at[pl.ds(s * ROWS_A, ROWS_A)])
    pltpu.sync_copy(z16_hbm.at[pl.ds(0, ROWS_T)], degt.at[pl.ds(s * ROWS_T, ROWS_T)])
    pltpu.sync_copy(ones_hbm, onesv)
    plsc.subcore_barrier()

    def deg_loop(dst_ref, nch, acc):
        def body(i, carry):
            base = pl.multiple_of((w * nch + i) * CH, CH)
            pltpu.sync_copy(dst_ref.at[pl.ds(base, CH)], didx)
            pltpu.sync_copy(onesv, acc.at[didx], add=True)
            return carry
        lax.fori_loop(0, nch, body, 0)

    deg_loop(aa_dst_hbm, CH_AA, dega)
    deg_loop(ta_dst_hbm, CH_AT, dega)
    deg_loop(at_dst_hbm, CH_AT, degt)

    def gbody(j, carry):
        base = pl.multiple_of(w * 320 + j * 80, 8)
        pltpu.sync_copy(lab_hbm.at[pl.ds(base, 80)], lidx)
        pltpu.async_copy(emb_tab_hbm.at[lidx], erows, sem).wait()
        pltpu.sync_copy(erows, emb_out.at[pl.ds(base, 80)])
        return carry
    lax.fori_loop(0, 4, gbody, 0)

    plsc.subcore_barrier()
    pltpu.sync_copy(dega.at[pl.ds(s * ROWS_A, ROWS_A)],
                    dega_out.at[c, pl.ds(s * ROWS_A, ROWS_A)])
    pltpu.sync_copy(degt.at[pl.ds(s * ROWS_T, ROWS_T)],
                    degt_out.at[c, pl.ds(s * ROWS_T, ROWS_T)])


@functools.partial(
    pl.kernel,
    mesh=_MESH,
    out_type=(
        jax.ShapeDtypeStruct((2, NPA, H), F32),
        jax.ShapeDtypeStruct((2, NPT, H), F32),
    ),
    scratch_types=[
        pltpu.VMEM((CH,), I32),
        pltpu.VMEM((CH,), I32),
        pltpu.VMEM((CH, H), F32),
        pltpu.VMEM_SHARED((NPA, H), F32),
        pltpu.VMEM_SHARED((NPT, H), F32),
        pltpu.SemaphoreType.DMA,
    ],
)
def _sc_seg(maa_hbm, mta_hbm, mat_hbm,
            aa_src_hbm, aa_dst_hbm, ta_src_hbm, ta_dst_hbm,
            at_src_hbm, at_dst_hbm, zrows_hbm,
            outa, outt,
            sidx, didx, rows, acca, acct, sem):
    c = lax.axis_index("c")
    s = lax.axis_index("s")
    w = c * 16 + s
    pltpu.sync_copy(zrows_hbm, acca.at[pl.ds(s * ROWS_A, ROWS_A)])
    pltpu.sync_copy(zrows_hbm.at[pl.ds(0, ROWS_T)], acct.at[pl.ds(s * ROWS_T, ROWS_T)])
    plsc.subcore_barrier()

    def seg_loop(src_ref, dst_ref, tab_ref, nch, acc):
        def body(i, carry):
            base = pl.multiple_of((w * nch + i) * CH, CH)
            pltpu.sync_copy(src_ref.at[pl.ds(base, CH)], sidx)
            pltpu.sync_copy(dst_ref.at[pl.ds(base, CH)], didx)
            pltpu.async_copy(tab_ref.at[sidx], rows, sem).wait()
            pltpu.sync_copy(rows, acc.at[didx], add=True)
            return carry
        lax.fori_loop(0, nch, body, 0)

    seg_loop(aa_src_hbm, aa_dst_hbm, maa_hbm, CH_AA, acca)
    seg_loop(ta_src_hbm, ta_dst_hbm, mta_hbm, CH_AT, acca)
    seg_loop(at_src_hbm, at_dst_hbm, mat_hbm, CH_AT, acct)

    plsc.subcore_barrier()
    pltpu.sync_copy(acca.at[pl.ds(s * ROWS_A, ROWS_A)],
                    outa.at[c, pl.ds(s * ROWS_A, ROWS_A)])
    pltpu.sync_copy(acct.at[pl.ds(s * ROWS_T, ROWS_T)],
                    outt.at[c, pl.ds(s * ROWS_T, ROWS_T)])


GB = 1000
GRID = N_AST // GB

_blk = lambda shape, imap: pl.BlockSpec(shape, imap)
_whole = lambda shape: pl.BlockSpec(shape, lambda i: tuple(0 for _ in shape))


def _tc_pre(emb, content, Wc, bc, temb, Waa, Wat, Wta):
    def body(emb_ref, cont_ref, Wc_ref, bc_ref, temb_ref, Waa_ref, Wat_ref,
             Wta_ref, maa_ref, mat_ref, mta_ref):
        h0 = jnp.concatenate(
            [emb_ref[...],
             jnp.dot(cont_ref[...], Wc_ref[...], preferred_element_type=F32)
             + bc_ref[...]], axis=1)
        maa_ref[...] = jnp.dot(h0, Waa_ref[...], preferred_element_type=F32)
        mat_ref[...] = jnp.dot(h0, Wat_ref[...], preferred_element_type=F32)

        @pl.when(pl.program_id(0) == 0)
        def _():
            row = jnp.dot(temb_ref[...], Wta_ref[...], preferred_element_type=F32)
            mta_ref[...] = jnp.broadcast_to(row, (N_TEST, H))

    return pl.pallas_call(
        body,
        grid=(GRID,),
        in_specs=[
            _blk((GB, 64), lambda i: (i, 0)),
            _blk((GB, H), lambda i: (i, 0)),
            _whole((H, 64)),
            _whole((1, 64)),
            _whole((1, H)),
            _whole((H, H)),
            _whole((H, H)),
            _whole((H, H)),
        ],
        out_specs=[
            _blk((GB, H), lambda i: (i, 0)),
            _blk((GB, H), lambda i: (i, 0)),
            _whole((N_TEST, H)),
        ],
        out_shape=[
            jax.ShapeDtypeStruct((N_AST, H), F32),
            jax.ShapeDtypeStruct((N_AST, H), F32),
            jax.ShapeDtypeStruct((N_TEST, H), F32),
        ],
    )(emb, content, Wc, bc, temb, Waa, Wat, Wta)


def _tc_combine(acca, acct, dega, degt, ba, bt, res, nxt, dec, emit_h):
    with_res = res is not None
    with_nxt = nxt is not None
    final = dec is not None

    def body(*refs):
        it = iter(refs)
        acca_ref = next(it); acct_ref = next(it)
        dega_ref = next(it); degt_ref = next(it)
        ba_ref = next(it); bt_ref = next(it)
        if with_res:
            resa_ref = next(it); rest_ref = next(it)
        if with_nxt:
            Waa_ref = next(it); Wat_ref = next(it); Wta_ref = next(it)
        if final:
            Wd_ref = next(it); bd_ref = next(it)
        if emit_h:
            ha_ref = next(it); ht_ref = next(it)
        if with_nxt:
            maa_ref = next(it); mat_ref = next(it); mta_ref = next(it)
        if final:
            lg_ref = next(it); pr_ref = next(it)

        agg = acca_ref[0] + acca_ref[1]
        deg = jnp.maximum(dega_ref[0, :, 0:1] + dega_ref[1, :, 0:1], 1.0)
        h = jnp.maximum(agg / deg + ba_ref[...], 0.0)
        if with_res:
            h = resa_ref[...] + h
        if emit_h:
            ha_ref[...] = h
        if with_nxt:
            maa_ref[...] = jnp.dot(h, Waa_ref[...], preferred_element_type=F32)
            mat_ref[...] = jnp.dot(h, Wat_ref[...], preferred_element_type=F32)
        if final:
            lg = jnp.dot(h, Wd_ref[...], preferred_element_type=F32) + bd_ref[...]
            lg_ref[...] = lg
            m = jnp.max(lg, axis=1, keepdims=True)
            e = jnp.exp(lg - m)
            pr_ref[...] = e / jnp.sum(e, axis=1, keepdims=True)

        @pl.when(pl.program_id(0) == 0)
        def _():
            agg_t = acct_ref[0, :N_TEST] + acct_ref[1, :N_TEST]
            deg_t = jnp.maximum(degt_ref[0, :N_TEST, 0:1]
                                + degt_ref[1, :N_TEST, 0:1], 1.0)
            ht = jnp.maximum(agg_t / deg_t + bt_ref[...], 0.0)
            if with_res:
                ht = rest_ref[...] + ht
            if emit_h:
                ht_ref[...] = ht
            if with_nxt:
                mta_ref[...] = jnp.dot(ht, Wta_ref[...], preferred_element_type=F32)

    in_specs = [
        _blk((2, GB, H), lambda i: (0, i, 0)),
        _whole((2, NPT, H)),
        _blk((2, GB, 16), lambda i: (0, i, 0)),
        _whole((2, NPT, 16)),
        _whole((1, H)),
        _whole((1, H)),
    ]
    args = [acca, acct, dega, degt, ba, bt]
    if with_res:
        in_specs += [_blk((GB, H), lambda i: (i, 0)), _whole((N_TEST, H))]
        args += [res[0], res[1]]
    if with_nxt:
        in_specs += [_whole((H, H))] * 3
        args += list(nxt)
    if final:
        in_specs += [_whole((H, 3)), _whole((1, 3))]
        args += list(dec)

    out_specs, out_shape = [], []
    if emit_h:
        out_specs += [_blk((GB, H), lambda i: (i, 0)), _whole((N_TEST, H))]
        out_shape += [jax.ShapeDtypeStruct((N_AST, H), F32),
                      jax.ShapeDtypeStruct((N_TEST, H), F32)]
    if with_nxt:
        out_specs += [_blk((GB, H), lambda i: (i, 0)),
                      _blk((GB, H), lambda i: (i, 0)),
                      _whole((N_TEST, H))]
        out_shape += [jax.ShapeDtypeStruct((N_AST, H), F32),
                      jax.ShapeDtypeStruct((N_AST, H), F32),
                      jax.ShapeDtypeStruct((N_TEST, H), F32)]
    if final:
        out_specs += [_blk((GB, 3), lambda i: (i, 0)),
                      _blk((GB, 3), lambda i: (i, 0))]
        out_shape += [jax.ShapeDtypeStruct((N_AST, 3), F32),
                      jax.ShapeDtypeStruct((N_AST, 3), F32)]

    return pl.pallas_call(
        body, grid=(GRID,), in_specs=in_specs, out_specs=out_specs,
        out_shape=out_shape)(*args)


def _pad_edges(src, dst, e, e_pad, trash):
    pad = e_pad - e
    return (jnp.concatenate([src, jnp.zeros((pad,), I32)]),
            jnp.concatenate([dst, jnp.full((pad,), trash, I32)]))


def kernel(ast_label, ast_content, astast_src, astast_dst, asttest_src,
           asttest_dst, testast_src, testast_dst, params):
    aa_src, aa_dst = _pad_edges(astast_src, astast_dst, E_AA, E_AA_P, N_AST)
    at_src, at_dst = _pad_edges(asttest_src, asttest_dst, E_AT, E_AT_P, N_TEST)
    ta_src, ta_dst = _pad_edges(testast_src, testast_dst, E_AT, E_AT_P, N_AST)
    lab = jnp.concatenate([ast_label, jnp.zeros((NPA - N_AST,), I32)])

    z16 = jnp.zeros((ROWS_A, 16), F32)
    ones16 = jnp.ones((CH, 16), F32)
    zrows = jnp.zeros((ROWS_A, H), F32)

    emb, dega, degt = _sc_init(lab, params["ast_label_emb"], aa_dst, ta_dst,
                               at_dst, z16, ones16)

    bc = params["ast_content_b"].reshape(1, 64)
    temb = params["test_embedding"].reshape(1, H)
    wl = lambda l: (params["l%d_W_astast" % l], params["l%d_W_asttest" % l],
                    params["l%d_W_testast" % l])
    bl = lambda l: (params["l%d_b_ast" % l].reshape(1, H),
                    params["l%d_b_test" % l].reshape(1, H))

    maa, mat, mta = _tc_pre(emb, ast_content, params["ast_content_W"], bc,
                            temb, *wl(1))

    def seg(maa, mat, mta):
        return _sc_seg(maa, mta, mat, aa_src, aa_dst, ta_src, ta_dst,
                       at_src, at_dst, zrows)

    acca, acct = seg(maa, mat, mta)
    b1a, b1t = bl(1)
    h1a, h1t, maa, mat, mta = _tc_combine(acca, acct, dega, degt, b1a, b1t,
                                          None, wl(2), None, True)
    acca, acct = seg(maa, mat, mta)
    b2a, b2t = bl(2)
    maa, mat, mta = _tc_combine(acca, acct, dega, degt, b2a, b2t,
                                (h1a, h1t), wl(3), None, False)
    acca, acct = seg(maa, mat, mta)
    b3a, b3t = bl(3)
    h3a, h3t, maa, mat, mta = _tc_combine(acca, acct, dega, degt, b3a, b3t,
                                          None, wl(4), None, True)
    acca, acct = seg(maa, mat, mta)
    b4a, b4t = bl(4)
    maa, mat, mta = _tc_combine(acca, acct, dega, degt, b4a, b4t,
                                (h3a, h3t), wl(5), None, False)
    acca, acct = seg(maa, mat, mta)
    b5a, b5t = bl(5)
    dec = (params["ast_dec_W"], params["ast_dec_b"].reshape(1, 3))
    h5a, h5t, logits, pred = _tc_combine(acca, acct, dega, degt, b5a, b5t,
                                         None, None, dec, True)
    return h5a, h5t, logits, pred


# speedup vs baseline: 2.5221x; 2.5221x over previous
import functools

import jax
import jax.numpy as jnp
from jax import lax
from jax.experimental import pallas as pl
from jax.experimental.pallas import tpu as pltpu
from jax.experimental.pallas import tpu_sc as plsc

F32 = jnp.float32
I32 = jnp.int32

N_AST = 10000
N_TEST = 500
H = 128
E_AA = 320000
E_AT = 100000

NPA = 10240
NPT = 512
NW = 32
CH = 128
CH_AA = 79
CH_AT = 25
E_AA_P = NW * CH_AA * CH
E_AT_P = NW * CH_AT * CH
ROWS_A = NPA // 16
ROWS_T = NPT // 16

_MESH = plsc.VectorSubcoreMesh(core_axis_name="c", subcore_axis_name="s")


@functools.partial(
    pl.kernel,
    mesh=_MESH,
    out_type=(
        jax.ShapeDtypeStruct((NPA, H), F32),
        jax.ShapeDtypeStruct((2, NPA, H), F32),
        jax.ShapeDtypeStruct((2, NPT, H), F32),
    ),
    scratch_types=[
        pltpu.VMEM((CH,), I32),
        pltpu.VMEM((CH, H), F32),
        pltpu.VMEM((80,), I32),
        pltpu.VMEM((80, H), F32),
        pltpu.VMEM_SHARED((NPA, H), F32),
        pltpu.VMEM_SHARED((NPT, H), F32),
        pltpu.SemaphoreType.DMA,
    ],
)
def _sc_init(lab_hbm, emb_tab_hbm, aa_dst_hbm, ta_dst_hbm, at_dst_hbm,
             zrows_hbm, ones_hbm,
             emb_out, dega_out, degt_out,
             didx, onesv, lidx, erows, dega, degt, sem):
    c = lax.axis_index("c")
    s = lax.axis_index("s")
    w = c * 16 + s
    pltpu.sync_copy(zrows_hbm, dega.at[pl.ds(s * ROWS_A, ROWS_A)])
    pltpu.sync_copy(zrows_hbm.at[pl.ds(0, ROWS_T)], degt.at[pl.ds(s * ROWS_T, ROWS_T)])
    pltpu.sync_copy(ones_hbm, onesv)
    plsc.subcore_barrier()

    def deg_loop(dst_ref, nch, acc):
        def body(i, carry):
            base = pl.multiple_of((w * nch + i) * CH, CH)
            pltpu.sync_copy(dst_ref.at[pl.ds(base, CH)], didx)
            pltpu.sync_copy(onesv, acc.at[didx], add=True)
            return carry
        lax.fori_loop(0, nch, body, 0)

    deg_loop(aa_dst_hbm, CH_AA, dega)
    deg_loop(ta_dst_hbm, CH_AT, dega)
    deg_loop(at_dst_hbm, CH_AT, degt)

    def gbody(j, carry):
        base = pl.multiple_of(w * 320 + j * 80, 8)
        pltpu.sync_copy(lab_hbm.at[pl.ds(base, 80)], lidx)
        pltpu.async_copy(emb_tab_hbm.at[lidx], erows, sem).wait()
        pltpu.sync_copy(erows, emb_out.at[pl.ds(base, 80)])
        return carry
    lax.fori_loop(0, 4, gbody, 0)

    plsc.subcore_barrier()
    pltpu.sync_copy(dega.at[pl.ds(s * ROWS_A, ROWS_A)],
                    dega_out.at[c, pl.ds(s * ROWS_A, ROWS_A)])
    pltpu.sync_copy(degt.at[pl.ds(s * ROWS_T, ROWS_T)],
                    degt_out.at[c, pl.ds(s * ROWS_T, ROWS_T)])


@functools.partial(
    pl.kernel,
    mesh=_MESH,
    out_type=(
        jax.ShapeDtypeStruct((2, NPA, H), F32),
        jax.ShapeDtypeStruct((2, NPT, H), F32),
    ),
    scratch_types=[
        pltpu.VMEM((CH,), I32),
        pltpu.VMEM((CH,), I32),
        pltpu.VMEM((CH, H), F32),
        pltpu.VMEM_SHARED((NPA, H), F32),
        pltpu.VMEM_SHARED((NPT, H), F32),
        pltpu.SemaphoreType.DMA,
    ],
)
def _sc_seg(maa_hbm, mta_hbm, mat_hbm,
            aa_src_hbm, aa_dst_hbm, ta_src_hbm, ta_dst_hbm,
            at_src_hbm, at_dst_hbm, zrows_hbm,
            outa, outt,
            sidx, didx, rows, acca, acct, sem):
    c = lax.axis_index("c")
    s = lax.axis_index("s")
    w = c * 16 + s
    pltpu.sync_copy(zrows_hbm, acca.at[pl.ds(s * ROWS_A, ROWS_A)])
    pltpu.sync_copy(zrows_hbm.at[pl.ds(0, ROWS_T)], acct.at[pl.ds(s * ROWS_T, ROWS_T)])
    plsc.subcore_barrier()

    def seg_loop(src_ref, dst_ref, tab_ref, nch, acc):
        def body(i, carry):
            base = pl.multiple_of((w * nch + i) * CH, CH)
            pltpu.sync_copy(src_ref.at[pl.ds(base, CH)], sidx)
            pltpu.sync_copy(dst_ref.at[pl.ds(base, CH)], didx)
            pltpu.async_copy(tab_ref.at[sidx], rows, sem).wait()
            pltpu.sync_copy(rows, acc.at[didx], add=True)
            return carry
        lax.fori_loop(0, nch, body, 0)

    seg_loop(aa_src_hbm, aa_dst_hbm, maa_hbm, CH_AA, acca)
    seg_loop(ta_src_hbm, ta_dst_hbm, mta_hbm, CH_AT, acca)
    seg_loop(at_src_hbm, at_dst_hbm, mat_hbm, CH_AT, acct)

    plsc.subcore_barrier()
    pltpu.sync_copy(acca.at[pl.ds(s * ROWS_A, ROWS_A)],
                    outa.at[c, pl.ds(s * ROWS_A, ROWS_A)])
    pltpu.sync_copy(acct.at[pl.ds(s * ROWS_T, ROWS_T)],
                    outt.at[c, pl.ds(s * ROWS_T, ROWS_T)])


GB = 1000
GRID = N_AST // GB

_blk = lambda shape, imap: pl.BlockSpec(shape, imap)
_whole = lambda shape: pl.BlockSpec(shape, lambda i: tuple(0 for _ in shape))


def _tc_pre(emb, content, Wc, bc, temb, Waa, Wat, Wta):
    def body(emb_ref, cont_ref, Wc_ref, bc_ref, temb_ref, Waa_ref, Wat_ref,
             Wta_ref, maa_ref, mat_ref, mta_ref):
        h0 = jnp.concatenate(
            [emb_ref[:, :64],
             jnp.dot(cont_ref[...], Wc_ref[...], preferred_element_type=F32)
             + bc_ref[...]], axis=1)
        maa_ref[...] = jnp.dot(h0, Waa_ref[...], preferred_element_type=F32)
        mat_ref[...] = jnp.dot(h0, Wat_ref[...], preferred_element_type=F32)

        @pl.when(pl.program_id(0) == 0)
        def _():
            row = jnp.dot(temb_ref[...], Wta_ref[...], preferred_element_type=F32)
            mta_ref[...] = jnp.broadcast_to(row, (N_TEST, H))

    return pl.pallas_call(
        body,
        grid=(GRID,),
        in_specs=[
            _blk((GB, H), lambda i: (i, 0)),
            _blk((GB, H), lambda i: (i, 0)),
            _whole((H, 64)),
            _whole((1, 64)),
            _whole((1, H)),
            _whole((H, H)),
            _whole((H, H)),
            _whole((H, H)),
        ],
        out_specs=[
            _blk((GB, H), lambda i: (i, 0)),
            _blk((GB, H), lambda i: (i, 0)),
            _whole((N_TEST, H)),
        ],
        out_shape=[
            jax.ShapeDtypeStruct((N_AST, H), F32),
            jax.ShapeDtypeStruct((N_AST, H), F32),
            jax.ShapeDtypeStruct((N_TEST, H), F32),
        ],
    )(emb, content, Wc, bc, temb, Waa, Wat, Wta)


def _tc_combine(acca, acct, dega, degt, ba, bt, res, nxt, dec, emit_h):
    with_res = res is not None
    with_nxt = nxt is not None
    final = dec is not None

    def body(*refs):
        it = iter(refs)
        acca_ref = next(it); acct_ref = next(it)
        dega_ref = next(it); degt_ref = next(it)
        ba_ref = next(it); bt_ref = next(it)
        if with_res:
            resa_ref = next(it); rest_ref = next(it)
        if with_nxt:
            Waa_ref = next(it); Wat_ref = next(it); Wta_ref = next(it)
        if final:
            Wd_ref = next(it); bd_ref = next(it)
        if emit_h:
            ha_ref = next(it); ht_ref = next(it)
        if with_nxt:
            maa_ref = next(it); mat_ref = next(it); mta_ref = next(it)
        if final:
            lg_ref = next(it); pr_ref = next(it)

        agg = acca_ref[0] + acca_ref[1]
        deg = jnp.maximum(dega_ref[0, :, 0:1] + dega_ref[1, :, 0:1], 1.0)
        h = jnp.maximum(agg / deg + ba_ref[...], 0.0)
        if with_res:
            h = resa_ref[...] + h
        if emit_h:
            ha_ref[...] = h
        if with_nxt:
            maa_ref[...] = jnp.dot(h, Waa_ref[...], preferred_element_type=F32)
            mat_ref[...] = jnp.dot(h, Wat_ref[...], preferred_element_type=F32)
        if final:
            lg = jnp.dot(h, Wd_ref[...], preferred_element_type=F32) + bd_ref[...]
            lg_ref[...] = lg
            m = jnp.max(lg, axis=1, keepdims=True)
            e = jnp.exp(lg - m)
            pr_ref[...] = e / jnp.sum(e, axis=1, keepdims=True)

        @pl.when(pl.program_id(0) == 0)
        def _():
            agg_t = acct_ref[0, :N_TEST] + acct_ref[1, :N_TEST]
            deg_t = jnp.maximum(degt_ref[0, :N_TEST, 0:1]
                                + degt_ref[1, :N_TEST, 0:1], 1.0)
            ht = jnp.maximum(agg_t / deg_t + bt_ref[...], 0.0)
            if with_res:
                ht = rest_ref[...] + ht
            if emit_h:
                ht_ref[...] = ht
            if with_nxt:
                mta_ref[...] = jnp.dot(ht, Wta_ref[...], preferred_element_type=F32)

    in_specs = [
        _blk((2, GB, H), lambda i: (0, i, 0)),
        _whole((2, NPT, H)),
        _blk((2, GB, H), lambda i: (0, i, 0)),
        _whole((2, NPT, H)),
        _whole((1, H)),
        _whole((1, H)),
    ]
    args = [acca, acct, dega, degt, ba, bt]
    if with_res:
        in_specs += [_blk((GB, H), lambda i: (i, 0)), _whole((N_TEST, H))]
        args += [res[0], res[1]]
    if with_nxt:
        in_specs += [_whole((H, H))] * 3
        args += list(nxt)
    if final:
        in_specs += [_whole((H, 3)), _whole((1, 3))]
        args += list(dec)

    out_specs, out_shape = [], []
    if emit_h:
        out_specs += [_blk((GB, H), lambda i: (i, 0)), _whole((N_TEST, H))]
        out_shape += [jax.ShapeDtypeStruct((N_AST, H), F32),
                      jax.ShapeDtypeStruct((N_TEST, H), F32)]
    if with_nxt:
        out_specs += [_blk((GB, H), lambda i: (i, 0)),
                      _blk((GB, H), lambda i: (i, 0)),
                      _whole((N_TEST, H))]
        out_shape += [jax.ShapeDtypeStruct((N_AST, H), F32),
                      jax.ShapeDtypeStruct((N_AST, H), F32),
                      jax.ShapeDtypeStruct((N_TEST, H), F32)]
    if final:
        out_specs += [_blk((GB, 3), lambda i: (i, 0)),
                      _blk((GB, 3), lambda i: (i, 0))]
        out_shape += [jax.ShapeDtypeStruct((N_AST, 3), F32),
                      jax.ShapeDtypeStruct((N_AST, 3), F32)]

    return pl.pallas_call(
        body, grid=(GRID,), in_specs=in_specs, out_specs=out_specs,
        out_shape=out_shape)(*args)


def _pad_edges(src, dst, e, e_pad, trash):
    pad = e_pad - e
    return (jnp.concatenate([src, jnp.zeros((pad,), I32)]),
            jnp.concatenate([dst, jnp.full((pad,), trash, I32)]))


def kernel(ast_label, ast_content, astast_src, astast_dst, asttest_src,
           asttest_dst, testast_src, testast_dst, params):
    aa_src, aa_dst = _pad_edges(astast_src, astast_dst, E_AA, E_AA_P, N_AST)
    at_src, at_dst = _pad_edges(asttest_src, asttest_dst, E_AT, E_AT_P, N_TEST)
    ta_src, ta_dst = _pad_edges(testast_src, testast_dst, E_AT, E_AT_P, N_AST)
    lab = jnp.concatenate([ast_label, jnp.zeros((NPA - N_AST,), I32)])

    ones128 = jnp.ones((CH, H), F32)
    zrows = jnp.zeros((ROWS_A, H), F32)
    emb_tab = jnp.pad(params["ast_label_emb"], ((0, 0), (0, H - 64)))

    emb, dega, degt = _sc_init(lab, emb_tab, aa_dst, ta_dst,
                               at_dst, zrows, ones128)

    bc = params["ast_content_b"].reshape(1, 64)
    temb = params["test_embedding"].reshape(1, H)
    wl = lambda l: (params["l%d_W_astast" % l], params["l%d_W_asttest" % l],
                    params["l%d_W_testast" % l])
    bl = lambda l: (params["l%d_b_ast" % l].reshape(1, H),
                    params["l%d_b_test" % l].reshape(1, H))

    maa, mat, mta = _tc_pre(emb, ast_content, params["ast_content_W"], bc,
                            temb, *wl(1))

    def seg(maa, mat, mta):
        return _sc_seg(maa, mta, mat, aa_src, aa_dst, ta_src, ta_dst,
                       at_src, at_dst, zrows)

    acca, acct = seg(maa, mat, mta)
    b1a, b1t = bl(1)
    h1a, h1t, maa, mat, mta = _tc_combine(acca, acct, dega, degt, b1a, b1t,
                                          None, wl(2), None, True)
    acca, acct = seg(maa, mat, mta)
    b2a, b2t = bl(2)
    maa, mat, mta = _tc_combine(acca, acct, dega, degt, b2a, b2t,
                                (h1a, h1t), wl(3), None, False)
    acca, acct = seg(maa, mat, mta)
    b3a, b3t = bl(3)
    h3a, h3t, maa, mat, mta = _tc_combine(acca, acct, dega, degt, b3a, b3t,
                                          None, wl(4), None, True)
    acca, acct = seg(maa, mat, mta)
    b4a, b4t = bl(4)
    maa, mat, mta = _tc_combine(acca, acct, dega, degt, b4a, b4t,
                                (h3a, h3t), wl(5), None, False)
    acca, acct = seg(maa, mat, mta)
    b5a, b5t = bl(5)
    dec = (params["ast_dec_W"], params["ast_dec_b"].reshape(1, 3))
    h5a, h5t, logits, pred = _tc_combine(acca, acct, dega, degt, b5a, b5t,
                                         None, None, dec, True)
    return h5a, h5t, logits, pred


# speedup vs baseline: 2.6450x; 1.0487x over previous
import functools

import jax
import jax.numpy as jnp
from jax import lax
from jax.experimental import pallas as pl
from jax.experimental.pallas import tpu as pltpu
from jax.experimental.pallas import tpu_sc as plsc

F32 = jnp.float32
I32 = jnp.int32

N_AST = 10000
N_TEST = 500
H = 128
E_AA = 320000
E_AT = 100000

NPA = 10240
NPT = 512
NW = 32
CH = 128
CH_AA = 80
CH_AT = 26
WS_AA = 80
WS_AT = 32
E_AA_P = NW * CH_AA * CH
E_AT_P = NW * CH_AT * CH
ROWS_A = NPA // 16
ROWS_T = NPT // 16

_MESH = plsc.VectorSubcoreMesh(core_axis_name="c", subcore_axis_name="s")


@functools.partial(
    pl.kernel,
    mesh=_MESH,
    out_type=(
        jax.ShapeDtypeStruct((NPA, H), F32),
        jax.ShapeDtypeStruct((2, NPA, H), F32),
        jax.ShapeDtypeStruct((2, NPT, H), F32),
    ),
    scratch_types=[
        pltpu.VMEM((WS_AA, CH), I32),
        pltpu.VMEM((CH, H), F32),
        pltpu.VMEM((80,), I32),
        pltpu.VMEM((80, H), F32),
        pltpu.VMEM_SHARED((NPA, H), F32),
        pltpu.VMEM_SHARED((NPT, H), F32),
        pltpu.SemaphoreType.DMA,
    ],
)
def _sc_init(lab_hbm, emb_tab_hbm, aa_dst_hbm, ta_dst_hbm, at_dst_hbm,
             zrows_hbm, ones_hbm,
             emb_out, dega_out, degt_out,
             dbuf, onesv, lidx, erows, dega, degt, sem):
    c = lax.axis_index("c")
    s = lax.axis_index("s")
    w = c * 16 + s
    pltpu.sync_copy(zrows_hbm, dega.at[pl.ds(s * ROWS_A, ROWS_A)])
    pltpu.sync_copy(zrows_hbm.at[pl.ds(0, ROWS_T)], degt.at[pl.ds(s * ROWS_T, ROWS_T)])
    pltpu.sync_copy(ones_hbm, onesv)
    plsc.subcore_barrier()

    def deg_loop(dst_ref, ws, nch, acc):
        pltpu.sync_copy(dst_ref.at[w], dbuf.at[pl.ds(0, ws)])

        def body(i, carry):
            pltpu.sync_copy(onesv, acc.at[dbuf.at[i]], add=True)
            return carry
        lax.fori_loop(0, nch, body, 0)

    deg_loop(aa_dst_hbm, WS_AA, CH_AA, dega)
    deg_loop(ta_dst_hbm, WS_AT, CH_AT, dega)
    deg_loop(at_dst_hbm, WS_AT, CH_AT, degt)

    def gbody(j, carry):
        base = pl.multiple_of(w * 320 + j * 80, 8)
        pltpu.sync_copy(lab_hbm.at[pl.ds(base, 80)], lidx)
        pltpu.async_copy(emb_tab_hbm.at[lidx], erows, sem).wait()
        pltpu.sync_copy(erows, emb_out.at[pl.ds(base, 80)])
        return carry
    lax.fori_loop(0, 4, gbody, 0)

    plsc.subcore_barrier()
    pltpu.sync_copy(dega.at[pl.ds(s * ROWS_A, ROWS_A)],
                    dega_out.at[c, pl.ds(s * ROWS_A, ROWS_A)])
    pltpu.sync_copy(degt.at[pl.ds(s * ROWS_T, ROWS_T)],
                    degt_out.at[c, pl.ds(s * ROWS_T, ROWS_T)])


@functools.partial(
    pl.kernel,
    mesh=_MESH,
    out_type=(
        jax.ShapeDtypeStruct((2, NPA, H), F32),
        jax.ShapeDtypeStruct((2, NPT, H), F32),
    ),
    scratch_types=[
        pltpu.VMEM((40, CH), I32),
        pltpu.VMEM((40, CH), I32),
        pltpu.VMEM((CH, H), F32),
        pltpu.VMEM((CH, H), F32),
        pltpu.VMEM_SHARED((NPA, H), F32),
        pltpu.VMEM_SHARED((NPT, H), F32),
        pltpu.SemaphoreType.DMA,
        pltpu.SemaphoreType.DMA,
    ],
)
def _sc_seg(maa_hbm, mta_hbm, mat_hbm,
            aa_src_hbm, aa_dst_hbm, ta_src_hbm, ta_dst_hbm,
            at_src_hbm, at_dst_hbm, zrows_hbm,
            outa, outt,
            sbuf, dbuf, rows_a, rows_b, acca, acct, sem_a, sem_b):
    c = lax.axis_index("c")
    s = lax.axis_index("s")
    w = c * 16 + s
    pltpu.sync_copy(zrows_hbm, acca.at[pl.ds(s * ROWS_A, ROWS_A)])
    pltpu.sync_copy(zrows_hbm.at[pl.ds(0, ROWS_T)], acct.at[pl.ds(s * ROWS_T, ROWS_T)])
    plsc.subcore_barrier()

    def seg_loop(src_ref, dst_ref, tab_ref, acc, stages):
        for off, ncopy, nproc in stages:
            pltpu.sync_copy(src_ref.at[w, pl.ds(off, ncopy)],
                            sbuf.at[pl.ds(0, ncopy)])
            pltpu.sync_copy(dst_ref.at[w, pl.ds(off, ncopy)],
                            dbuf.at[pl.ds(0, ncopy)])
            pltpu.async_copy(tab_ref.at[sbuf.at[0]], rows_a, sem_a)
            npair = nproc // 2

            def body(i, carry):
                a = 2 * i
                b = a + 1
                pltpu.async_copy(tab_ref.at[sbuf.at[b]], rows_b, sem_b)
                pltpu.make_async_copy(tab_ref.at[sbuf.at[a]], rows_a, sem_a).wait()
                pltpu.sync_copy(rows_a, acc.at[dbuf.at[a]], add=True)

                @pl.when(i < npair - 1)
                def _():
                    pltpu.async_copy(tab_ref.at[sbuf.at[a + 2]], rows_a, sem_a)

                pltpu.make_async_copy(tab_ref.at[sbuf.at[b]], rows_b, sem_b).wait()
                pltpu.sync_copy(rows_b, acc.at[dbuf.at[b]], add=True)
                return carry
            lax.fori_loop(0, npair, body, 0)

    seg_loop(aa_src_hbm, aa_dst_hbm, maa_hbm, acca, [(0, 40, 40), (40, 40, 40)])
    seg_loop(ta_src_hbm, ta_dst_hbm, mta_hbm, acca, [(0, WS_AT, CH_AT)])
    seg_loop(at_src_hbm, at_dst_hbm, mat_hbm, acct, [(0, WS_AT, CH_AT)])

    plsc.subcore_barrier()
    pltpu.sync_copy(acca.at[pl.ds(s * ROWS_A, ROWS_A)],
                    outa.at[c, pl.ds(s * ROWS_A, ROWS_A)])
    pltpu.sync_copy(acct.at[pl.ds(s * ROWS_T, ROWS_T)],
                    outt.at[c, pl.ds(s * ROWS_T, ROWS_T)])


GB = 1000
GRID = N_AST // GB

_blk = lambda shape, imap: pl.BlockSpec(shape, imap)
_whole = lambda shape: pl.BlockSpec(shape, lambda i: tuple(0 for _ in shape))


def _tc_pre(emb, content, Wc, bc, temb, Waa, Wat, Wta):
    def body(emb_ref, cont_ref, Wc_ref, bc_ref, temb_ref, Waa_ref, Wat_ref,
             Wta_ref, maa_ref, mat_ref, mta_ref):
        h0 = jnp.concatenate(
            [emb_ref[:, :64],
             jnp.dot(cont_ref[...], Wc_ref[...], preferred_element_type=F32)
             + bc_ref[...]], axis=1)
        maa_ref[...] = jnp.dot(h0, Waa_ref[...], preferred_element_type=F32)
        mat_ref[...] = jnp.dot(h0, Wat_ref[...], preferred_element_type=F32)

        @pl.when(pl.program_id(0) == 0)
        def _():
            row = jnp.dot(temb_ref[...], Wta_ref[...], preferred_element_type=F32)
            mta_ref[...] = jnp.broadcast_to(row, (N_TEST, H))

    return pl.pallas_call(
        body,
        grid=(GRID,),
        in_specs=[
            _blk((GB, H), lambda i: (i, 0)),
            _blk((GB, H), lambda i: (i, 0)),
            _whole((H, 64)),
            _whole((1, 64)),
            _whole((1, H)),
            _whole((H, H)),
            _whole((H, H)),
            _whole((H, H)),
        ],
        out_specs=[
            _blk((GB, H), lambda i: (i, 0)),
            _blk((GB, H), lambda i: (i, 0)),
            _whole((N_TEST, H)),
        ],
        out_shape=[
            jax.ShapeDtypeStruct((N_AST, H), F32),
            jax.ShapeDtypeStruct((N_AST, H), F32),
            jax.ShapeDtypeStruct((N_TEST, H), F32),
        ],
    )(emb, content, Wc, bc, temb, Waa, Wat, Wta)


def _tc_combine(acca, acct, dega, degt, ba, bt, res, nxt, dec, emit_h):
    with_res = res is not None
    with_nxt = nxt is not None
    final = dec is not None

    def body(*refs):
        it = iter(refs)
        acca_ref = next(it); acct_ref = next(it)
        dega_ref = next(it); degt_ref = next(it)
        ba_ref = next(it); bt_ref = next(it)
        if with_res:
            resa_ref = next(it); rest_ref = next(it)
        if with_nxt:
            Waa_ref = next(it); Wat_ref = next(it); Wta_ref = next(it)
        if final:
            Wd_ref = next(it); bd_ref = next(it)
        if emit_h:
            ha_ref = next(it); ht_ref = next(it)
        if with_nxt:
            maa_ref = next(it); mat_ref = next(it); mta_ref = next(it)
        if final:
            lg_ref = next(it); pr_ref = next(it)

        agg = acca_ref[0] + acca_ref[1]
        deg = jnp.maximum(dega_ref[0, :, 0:1] + dega_ref[1, :, 0:1], 1.0)
        h = jnp.maximum(agg / deg + ba_ref[...], 0.0)
        if with_res:
            h = resa_ref[...] + h
        if emit_h:
            ha_ref[...] = h
        if with_nxt:
            maa_ref[...] = jnp.dot(h, Waa_ref[...], preferred_element_type=F32)
            mat_ref[...] = jnp.dot(h, Wat_ref[...], preferred_element_type=F32)
        if final:
            lg = jnp.dot(h, Wd_ref[...], preferred_element_type=F32) + bd_ref[...]
            lg_ref[...] = lg
            m = jnp.max(lg, axis=1, keepdims=True)
            e = jnp.exp(lg - m)
            pr_ref[...] = e / jnp.sum(e, axis=1, keepdims=True)

        @pl.when(pl.program_id(0) == 0)
        def _():
            agg_t = acct_ref[0, :N_TEST] + acct_ref[1, :N_TEST]
            deg_t = jnp.maximum(degt_ref[0, :N_TEST, 0:1]
                                + degt_ref[1, :N_TEST, 0:1], 1.0)
            ht = jnp.maximum(agg_t / deg_t + bt_ref[...], 0.0)
            if with_res:
                ht = rest_ref[...] + ht
            if emit_h:
                ht_ref[...] = ht
            if with_nxt:
                mta_ref[...] = jnp.dot(ht, Wta_ref[...], preferred_element_type=F32)

    in_specs = [
        _blk((2, GB, H), lambda i: (0, i, 0)),
        _whole((2, NPT, H)),
        _blk((2, GB, H), lambda i: (0, i, 0)),
        _whole((2, NPT, H)),
        _whole((1, H)),
        _whole((1, H)),
    ]
    args = [acca, acct, dega, degt, ba, bt]
    if with_res:
        in_specs += [_blk((GB, H), lambda i: (i, 0)), _whole((N_TEST, H))]
        args += [res[0], res[1]]
    if with_nxt:
        in_specs += [_whole((H, H))] * 3
        args += list(nxt)
    if final:
        in_specs += [_whole((H, 3)), _whole((1, 3))]
        args += list(dec)

    out_specs, out_shape = [], []
    if emit_h:
        out_specs += [_blk((GB, H), lambda i: (i, 0)), _whole((N_TEST, H))]
        out_shape += [jax.ShapeDtypeStruct((N_AST, H), F32),
                      jax.ShapeDtypeStruct((N_TEST, H), F32)]
    if with_nxt:
        out_specs += [_blk((GB, H), lambda i: (i, 0)),
                      _blk((GB, H), lambda i: (i, 0)),
                      _whole((N_TEST, H))]
        out_shape += [jax.ShapeDtypeStruct((N_AST, H), F32),
                      jax.ShapeDtypeStruct((N_AST, H), F32),
                      jax.ShapeDtypeStruct((N_TEST, H), F32)]
    if final:
        out_specs += [_blk((GB, 3), lambda i: (i, 0)),
                      _blk((GB, 3), lambda i: (i, 0))]
        out_shape += [jax.ShapeDtypeStruct((N_AST, 3), F32),
                      jax.ShapeDtypeStruct((N_AST, 3), F32)]

    return pl.pallas_call(
        body, grid=(GRID,), in_specs=in_specs, out_specs=out_specs,
        out_shape=out_shape)(*args)


def _pad_edges(src, dst, e, nch, ws, trash, trash_n):
    pad = NW * nch * CH - e
    tr = trash + (jnp.arange(pad, dtype=I32) % trash_n)
    s2 = jnp.concatenate([src, jnp.zeros((pad,), I32)]).reshape(NW, nch, CH)
    d2 = jnp.concatenate([dst, tr]).reshape(NW, nch, CH)
    if ws != nch:
        z = ((0, 0), (0, ws - nch), (0, 0))
        s2, d2 = jnp.pad(s2, z), jnp.pad(d2, z)
    return s2, d2


def kernel(ast_label, ast_content, astast_src, astast_dst, asttest_src,
           asttest_dst, testast_src, testast_dst, params):
    aa_src, aa_dst = _pad_edges(astast_src, astast_dst, E_AA, CH_AA, WS_AA,
                                N_AST, NPA - N_AST)
    at_src, at_dst = _pad_edges(asttest_src, asttest_dst, E_AT, CH_AT, WS_AT,
                                N_TEST, NPT - N_TEST)
    ta_src, ta_dst = _pad_edges(testast_src, testast_dst, E_AT, CH_AT, WS_AT,
                                N_AST, NPA - N_AST)
    lab = jnp.concatenate([ast_label, jnp.zeros((NPA - N_AST,), I32)])

    ones128 = jnp.ones((CH, H), F32)
    zrows = jnp.zeros((ROWS_A, H), F32)
    emb_tab = jnp.pad(params["ast_label_emb"], ((0, 0), (0, H - 64)))

    emb, dega, degt = _sc_init(lab, emb_tab, aa_dst, ta_dst,
                               at_dst, zrows, ones128)

    bc = params["ast_content_b"].reshape(1, 64)
    temb = params["test_embedding"].reshape(1, H)
    wl = lambda l: (params["l%d_W_astast" % l], params["l%d_W_asttest" % l],
                    params["l%d_W_testast" % l])
    bl = lambda l: (params["l%d_b_ast" % l].reshape(1, H),
                    params["l%d_b_test" % l].reshape(1, H))

    maa, mat, mta = _tc_pre(emb, ast_content, params["ast_content_W"], bc,
                            temb, *wl(1))

    def seg(maa, mat, mta):
        return _sc_seg(maa, mta, mat, aa_src, aa_dst, ta_src, ta_dst,
                       at_src, at_dst, zrows)

    acca, acct = seg(maa, mat, mta)
    b1a, b1t = bl(1)
    h1a, h1t, maa, mat, mta = _tc_combine(acca, acct, dega, degt, b1a, b1t,
                                          None, wl(2), None, True)
    acca, acct = seg(maa, mat, mta)
    b2a, b2t = bl(2)
    maa, mat, mta = _tc_combine(acca, acct, dega, degt, b2a, b2t,
                                (h1a, h1t), wl(3), None, False)
    acca, acct = seg(maa, mat, mta)
    b3a, b3t = bl(3)
    h3a, h3t, maa, mat, mta = _tc_combine(acca, acct, dega, degt, b3a, b3t,
                                          None, wl(4), None, True)
    acca, acct = seg(maa, mat, mta)
    b4a, b4t = bl(4)
    maa, mat, mta = _tc_combine(acca, acct, dega, degt, b4a, b4t,
                                (h3a, h3t), wl(5), None, False)
    acca, acct = seg(maa, mat, mta)
    b5a, b5t = bl(5)
    dec = (params["ast_dec_W"], params["ast_dec_b"].reshape(1, 3))
    h5a, h5t, logits, pred = _tc_combine(acca, acct, dega, degt, b5a, b5t,
                                         None, None, dec, True)
    return h5a, h5t, logits, pred


# speedup vs baseline: 2.8001x; 1.0586x over previous
import functools

import jax
import jax.numpy as jnp
from jax import lax
from jax.experimental import pallas as pl
from jax.experimental.pallas import tpu as pltpu
from jax.experimental.pallas import tpu_sc as plsc

F32 = jnp.float32
I32 = jnp.int32

N_AST = 10000
N_TEST = 500
H = 128
E_AA = 320000
E_AT = 100000

NPA = 10240
NPT = 512
NW = 32
CH = 128
AA0, AA1 = 128, 32
AT0, AT1 = 40, 10
WS_AT1 = 16
E_AA_P = 16 * (AA0 + AA1) * CH
E_AT_P = 16 * (AT0 + AT1) * CH
AA_ST0 = ((0, 40, 40), (40, 40, 40), (80, 40, 40), (120, 8, 8))
AA_ST1 = ((0, 32, 32),)
AT_ST0 = ((0, 40, 40),)
AT_ST1 = ((0, WS_AT1, AT1),)
ROWS_A = NPA // 16
ROWS_T = NPT // 16

_MESH = plsc.VectorSubcoreMesh(core_axis_name="c", subcore_axis_name="s")


@functools.partial(
    pl.kernel,
    mesh=_MESH,
    out_type=(
        jax.ShapeDtypeStruct((NPA, H), F32),
        jax.ShapeDtypeStruct((2, NPA, H), F32),
        jax.ShapeDtypeStruct((2, NPT, H), F32),
    ),
    scratch_types=[
        pltpu.VMEM((40, CH), I32),
        pltpu.VMEM((CH, H), F32),
        pltpu.VMEM((80,), I32),
        pltpu.VMEM((80, H), F32),
        pltpu.VMEM_SHARED((NPA, H), F32),
        pltpu.VMEM_SHARED((NPT, H), F32),
        pltpu.SemaphoreType.DMA,
    ],
)
def _sc_init(lab_hbm, emb_tab_hbm, aa_d0, aa_d1, ta_d0, ta_d1, at_d0, at_d1,
             zrows_hbm, ones_hbm,
             emb_out, dega_out, degt_out,
             dbuf, onesv, lidx, erows, dega, degt, sem):
    c = lax.axis_index("c")
    s = lax.axis_index("s")
    w = c * 16 + s
    pltpu.sync_copy(zrows_hbm, dega.at[pl.ds(s * ROWS_A, ROWS_A)])
    pltpu.sync_copy(zrows_hbm.at[pl.ds(0, ROWS_T)], degt.at[pl.ds(s * ROWS_T, ROWS_T)])
    pltpu.sync_copy(ones_hbm, onesv)
    plsc.subcore_barrier()

    def deg_loop(dst_ref, stages, acc):
        for off, ncopy, nproc in stages:
            pltpu.sync_copy(dst_ref.at[s, pl.ds(off, ncopy)],
                            dbuf.at[pl.ds(0, ncopy)])

            def body(i, carry):
                pltpu.sync_copy(onesv, acc.at[dbuf.at[i]], add=True)
                return carry
            lax.fori_loop(0, nproc, body, 0)

    @pl.when(c == 0)
    def _():
        deg_loop(aa_d0, AA_ST0, dega)
        deg_loop(ta_d0, AT_ST0, dega)
        deg_loop(at_d0, AT_ST0, degt)

    @pl.when(c == 1)
    def _():
        deg_loop(aa_d1, AA_ST1, dega)
        deg_loop(ta_d1, AT_ST1, dega)
        deg_loop(at_d1, AT_ST1, degt)

    def gbody(j, carry):
        base = pl.multiple_of(w * 320 + j * 80, 8)
        pltpu.sync_copy(lab_hbm.at[pl.ds(base, 80)], lidx)
        pltpu.async_copy(emb_tab_hbm.at[lidx], erows, sem).wait()
        pltpu.sync_copy(erows, emb_out.at[pl.ds(base, 80)])
        return carry
    lax.fori_loop(0, 4, gbody, 0)

    plsc.subcore_barrier()
    pltpu.sync_copy(dega.at[pl.ds(s * ROWS_A, ROWS_A)],
                    dega_out.at[c, pl.ds(s * ROWS_A, ROWS_A)])
    pltpu.sync_copy(degt.at[pl.ds(s * ROWS_T, ROWS_T)],
                    degt_out.at[c, pl.ds(s * ROWS_T, ROWS_T)])


@functools.partial(
    pl.kernel,
    mesh=_MESH,
    out_type=(
        jax.ShapeDtypeStruct((2, NPA, H), F32),
        jax.ShapeDtypeStruct((2, NPT, H), F32),
    ),
    scratch_types=[
        pltpu.VMEM((40, CH), I32),
        pltpu.VMEM((40, CH), I32),
        pltpu.VMEM((CH, H), F32),
        pltpu.VMEM((CH, H), F32),
        pltpu.VMEM_SHARED((NPA, H), F32),
        pltpu.VMEM_SHARED((NPT, H), F32),
        pltpu.SemaphoreType.DMA,
        pltpu.SemaphoreType.DMA,
    ],
)
def _sc_seg(maa_hbm, mta_hbm, mat_hbm,
            aa_s0, aa_d0, aa_s1, aa_d1, ta_s0, ta_d0, ta_s1, ta_d1,
            at_s0, at_d0, at_s1, at_d1, zrows_hbm,
            outa, outt,
            sbuf, dbuf, rows_a, rows_b, acca, acct, sem_a, sem_b):
    c = lax.axis_index("c")
    s = lax.axis_index("s")
    pltpu.sync_copy(zrows_hbm, acca.at[pl.ds(s * ROWS_A, ROWS_A)])
    pltpu.sync_copy(zrows_hbm.at[pl.ds(0, ROWS_T)], acct.at[pl.ds(s * ROWS_T, ROWS_T)])
    plsc.subcore_barrier()

    def seg_loop(src_ref, dst_ref, tab_ref, acc, stages):
        for off, ncopy, nproc in stages:
            pltpu.sync_copy(src_ref.at[s, pl.ds(off, ncopy)],
                            sbuf.at[pl.ds(0, ncopy)])
            pltpu.sync_copy(dst_ref.at[s, pl.ds(off, ncopy)],
                            dbuf.at[pl.ds(0, ncopy)])
            pltpu.async_copy(tab_ref.at[sbuf.at[0]], rows_a, sem_a)
            npair = nproc // 2

            def body(i, carry):
                a = 2 * i
                b = a + 1
                pltpu.async_copy(tab_ref.at[sbuf.at[b]], rows_b, sem_b)
                pltpu.make_async_copy(tab_ref.at[sbuf.at[a]], rows_a, sem_a).wait()
                pltpu.sync_copy(rows_a, acc.at[dbuf.at[a]], add=True)

                @pl.when(i < npair - 1)
                def _():
                    pltpu.async_copy(tab_ref.at[sbuf.at[a + 2]], rows_a, sem_a)

                pltpu.make_async_copy(tab_ref.at[sbuf.at[b]], rows_b, sem_b).wait()
                pltpu.sync_copy(rows_b, acc.at[dbuf.at[b]], add=True)
                return carry
            lax.fori_loop(0, npair, body, 0)

    @pl.when(c == 0)
    def _():
        seg_loop(aa_s0, aa_d0, maa_hbm, acca, AA_ST0)
        seg_loop(ta_s0, ta_d0, mta_hbm, acca, AT_ST0)
        seg_loop(at_s0, at_d0, mat_hbm, acct, AT_ST0)

    @pl.when(c == 1)
    def _():
        seg_loop(aa_s1, aa_d1, maa_hbm, acca, AA_ST1)
        seg_loop(ta_s1, ta_d1, mta_hbm, acca, AT_ST1)
        seg_loop(at_s1, at_d1, mat_hbm, acct, AT_ST1)

    plsc.subcore_barrier()
    pltpu.sync_copy(acca.at[pl.ds(s * ROWS_A, ROWS_A)],
                    outa.at[c, pl.ds(s * ROWS_A, ROWS_A)])
    pltpu.sync_copy(acct.at[pl.ds(s * ROWS_T, ROWS_T)],
                    outt.at[c, pl.ds(s * ROWS_T, ROWS_T)])


GB = 1000
GRID = N_AST // GB

_blk = lambda shape, imap: pl.BlockSpec(shape, imap)
_whole = lambda shape: pl.BlockSpec(shape, lambda i: tuple(0 for _ in shape))


def _tc_pre(emb, content, Wc, bc, temb, Waa, Wat, Wta):
    def body(emb_ref, cont_ref, Wc_ref, bc_ref, temb_ref, Waa_ref, Wat_ref,
             Wta_ref, maa_ref, mat_ref, mta_ref):
        h0 = jnp.concatenate(
            [emb_ref[:, :64],
             jnp.dot(cont_ref[...], Wc_ref[...], preferred_element_type=F32)
             + bc_ref[...]], axis=1)
        maa_ref[...] = jnp.dot(h0, Waa_ref[...], preferred_element_type=F32)
        mat_ref[...] = jnp.dot(h0, Wat_ref[...], preferred_element_type=F32)

        @pl.when(pl.program_id(0) == 0)
        def _():
            row = jnp.dot(temb_ref[...], Wta_ref[...], preferred_element_type=F32)
            mta_ref[...] = jnp.broadcast_to(row, (N_TEST, H))

    return pl.pallas_call(
        body,
        grid=(GRID,),
        in_specs=[
            _blk((GB, H), lambda i: (i, 0)),
            _blk((GB, H), lambda i: (i, 0)),
            _whole((H, 64)),
            _whole((1, 64)),
            _whole((1, H)),
            _whole((H, H)),
            _whole((H, H)),
            _whole((H, H)),
        ],
        out_specs=[
            _blk((GB, H), lambda i: (i, 0)),
            _blk((GB, H), lambda i: (i, 0)),
            _whole((N_TEST, H)),
        ],
        out_shape=[
            jax.ShapeDtypeStruct((N_AST, H), F32),
            jax.ShapeDtypeStruct((N_AST, H), F32),
            jax.ShapeDtypeStruct((N_TEST, H), F32),
        ],
    )(emb, content, Wc, bc, temb, Waa, Wat, Wta)


def _tc_combine(acca, acct, dega, degt, ba, bt, res, nxt, dec, emit_h):
    with_res = res is not None
    with_nxt = nxt is not None
    final = dec is not None

    def body(*refs):
        it = iter(refs)
        acca_ref = next(it); acct_ref = next(it)
        dega_ref = next(it); degt_ref = next(it)
        ba_ref = next(it); bt_ref = next(it)
        if with_res:
            resa_ref = next(it); rest_ref = next(it)
        if with_nxt:
            Waa_ref = next(it); Wat_ref = next(it); Wta_ref = next(it)
        if final:
            Wd_ref = next(it); bd_ref = next(it)
        if emit_h:
            ha_ref = next(it); ht_ref = next(it)
        if with_nxt:
            maa_ref = next(it); mat_ref = next(it); mta_ref = next(it)
        if final:
            lg_ref = next(it); pr_ref = next(it)

        agg = acca_ref[0] + acca_ref[1]
        deg = jnp.maximum(dega_ref[0, :, 0:1] + dega_ref[1, :, 0:1], 1.0)
        h = jnp.maximum(agg / deg + ba_ref[...], 0.0)
        if with_res:
            h = resa_ref[...] + h
        if emit_h:
            ha_ref[...] = h
        if with_nxt:
            maa_ref[...] = jnp.dot(h, Waa_ref[...], preferred_element_type=F32)
            mat_ref[...] = jnp.dot(h, Wat_ref[...], preferred_element_type=F32)
        if final:
            lg = jnp.dot(h, Wd_ref[...], preferred_element_type=F32) + bd_ref[...]
            lg_ref[...] = lg
            m = jnp.max(lg, axis=1, keepdims=True)
            e = jnp.exp(lg - m)
            pr_ref[...] = e / jnp.sum(e, axis=1, keepdims=True)

        @pl.when(pl.program_id(0) == 0)
        def _():
            agg_t = acct_ref[0, :N_TEST] + acct_ref[1, :N_TEST]
            deg_t = jnp.maximum(degt_ref[0, :N_TEST, 0:1]
                                + degt_ref[1, :N_TEST, 0:1], 1.0)
            ht = jnp.maximum(agg_t / deg_t + bt_ref[...], 0.0)
            if with_res:
                ht = rest_ref[...] + ht
            if emit_h:
                ht_ref[...] = ht
            if with_nxt:
                mta_ref[...] = jnp.dot(ht, Wta_ref[...], preferred_element_type=F32)

    in_specs = [
        _blk((2, GB, H), lambda i: (0, i, 0)),
        _whole((2, NPT, H)),
        _blk((2, GB, H), lambda i: (0, i, 0)),
        _whole((2, NPT, H)),
        _whole((1, H)),
        _whole((1, H)),
    ]
    args = [acca, acct, dega, degt, ba, bt]
    if with_res:
        in_specs += [_blk((GB, H), lambda i: (i, 0)), _whole((N_TEST, H))]
        args += [res[0], res[1]]
    if with_nxt:
        in_specs += [_whole((H, H))] * 3
        args += list(nxt)
    if final:
        in_specs += [_whole((H, 3)), _whole((1, 3))]
        args += list(dec)

    out_specs, out_shape = [], []
    if emit_h:
        out_specs += [_blk((GB, H), lambda i: (i, 0)), _whole((N_TEST, H))]
        out_shape += [jax.ShapeDtypeStruct((N_AST, H), F32),
                      jax.ShapeDtypeStruct((N_TEST, H), F32)]
    if with_nxt:
        out_specs += [_blk((GB, H), lambda i: (i, 0)),
                      _blk((GB, H), lambda i: (i, 0)),
                      _whole((N_TEST, H))]
        out_shape += [jax.ShapeDtypeStruct((N_AST, H), F32),
                      jax.ShapeDtypeStruct((N_AST, H), F32),
                      jax.ShapeDtypeStruct((N_TEST, H), F32)]
    if final:
        out_specs += [_blk((GB, 3), lambda i: (i, 0)),
                      _blk((GB, 3), lambda i: (i, 0))]
        out_shape += [jax.ShapeDtypeStruct((N_AST, 3), F32),
                      jax.ShapeDtypeStruct((N_AST, 3), F32)]

    return pl.pallas_call(
        body, grid=(GRID,), in_specs=in_specs, out_specs=out_specs,
        out_shape=out_shape)(*args)


def _pad_edges(src, dst, e, r0, r1, ws1, trash, trash_n):
    e_pad = 16 * (r0 + r1) * CH
    pad = e_pad - e
    tr = trash + (jnp.arange(pad, dtype=I32) % trash_n)
    s1d = jnp.concatenate([src, jnp.zeros((pad,), I32)])
    d1d = jnp.concatenate([dst, tr])
    cut = 16 * r0 * CH
    out = []
    for a in (s1d, d1d):
        a0 = a[:cut].reshape(16, r0, CH)
        a1 = a[cut:].reshape(16, r1, CH)
        if ws1 != r1:
            a1 = jnp.pad(a1, ((0, 0), (0, ws1 - r1), (0, 0)))
        out += [a0, a1]
    return out


def kernel(ast_label, ast_content, astast_src, astast_dst, asttest_src,
           asttest_dst, testast_src, testast_dst, params):
    aa_s0, aa_s1, aa_d0, aa_d1 = _pad_edges(
        astast_src, astast_dst, E_AA, AA0, AA1, AA1, N_AST, NPA - N_AST)
    at_s0, at_s1, at_d0, at_d1 = _pad_edges(
        asttest_src, asttest_dst, E_AT, AT0, AT1, WS_AT1, N_TEST, NPT - N_TEST)
    ta_s0, ta_s1, ta_d0, ta_d1 = _pad_edges(
        testast_src, testast_dst, E_AT, AT0, AT1, WS_AT1, N_AST, NPA - N_AST)
    lab = jnp.concatenate([ast_label, jnp.zeros((NPA - N_AST,), I32)])

    ones128 = jnp.ones((CH, H), F32)
    zrows = jnp.zeros((ROWS_A, H), F32)
    emb_tab = jnp.pad(params["ast_label_emb"], ((0, 0), (0, H - 64)))

    emb, dega, degt = _sc_init(lab, emb_tab, aa_d0, aa_d1, ta_d0, ta_d1,
                               at_d0, at_d1, zrows, ones128)

    bc = params["ast_content_b"].reshape(1, 64)
    temb = params["test_embedding"].reshape(1, H)
    wl = lambda l: (params["l%d_W_astast" % l], params["l%d_W_asttest" % l],
                    params["l%d_W_testast" % l])
    bl = lambda l: (params["l%d_b_ast" % l].reshape(1, H),
                    params["l%d_b_test" % l].reshape(1, H))

    maa, mat, mta = _tc_pre(emb, ast_content, params["ast_content_W"], bc,
                            temb, *wl(1))

    def seg(maa, mat, mta):
        return _sc_seg(maa, mta, mat, aa_s0, aa_d0, aa_s1, aa_d1,
                       ta_s0, ta_d0, ta_s1, ta_d1,
                       at_s0, at_d0, at_s1, at_d1, zrows)

    acca, acct = seg(maa, mat, mta)
    b1a, b1t = bl(1)
    h1a, h1t, maa, mat, mta = _tc_combine(acca, acct, dega, degt, b1a, b1t,
                                          None, wl(2), None, True)
    acca, acct = seg(maa, mat, mta)
    b2a, b2t = bl(2)
    maa, mat, mta = _tc_combine(acca, acct, dega, degt, b2a, b2t,
                                (h1a, h1t), wl(3), None, False)
    acca, acct = seg(maa, mat, mta)
    b3a, b3t = bl(3)
    h3a, h3t, maa, mat, mta = _tc_combine(acca, acct, dega, degt, b3a, b3t,
                                          None, wl(4), None, True)
    acca, acct = seg(maa, mat, mta)
    b4a, b4t = bl(4)
    maa, mat, mta = _tc_combine(acca, acct, dega, degt, b4a, b4t,
                                (h3a, h3t), wl(5), None, False)
    acca, acct = seg(maa, mat, mta)
    b5a, b5t = bl(5)
    dec = (params["ast_dec_W"], params["ast_dec_b"].reshape(1, 3))
    h5a, h5t, logits, pred = _tc_combine(acca, acct, dega, degt, b5a, b5t,
                                         None, None, dec, True)
    return h5a, h5t, logits, pred


# speedup vs baseline: 3.7876x; 1.3527x over previous
import functools

import jax
import jax.numpy as jnp
from jax import lax
from jax.experimental import pallas as pl
from jax.experimental.pallas import tpu as pltpu
from jax.experimental.pallas import tpu_sc as plsc

F32 = jnp.float32
I32 = jnp.int32

N_AST = 10000
N_TEST = 500
H = 128
E_AA = 320000
E_AT = 100000

NPA = 10240
NPT = 512
NW = 32
CH = 128
AA0, AA1 = 144, 16
AT0, AT1 = 40, 10
WS_AT1 = 16
E_AA_P = 16 * (AA0 + AA1) * CH
E_AT_P = 16 * (AT0 + AT1) * CH
AA_ST0 = ((0, 40, 40), (40, 40, 40), (80, 40, 40), (120, 24, 24))
AA_ST1 = ((0, 16, 16),)
AT_ST0 = ((0, 40, 40),)
AT_ST1 = ((0, WS_AT1, AT1),)
ROWS_A = NPA // 16
ROWS_T = NPT // 16

_MESH = plsc.VectorSubcoreMesh(core_axis_name="c", subcore_axis_name="s")


@functools.partial(
    pl.kernel,
    mesh=_MESH,
    out_type=(
        jax.ShapeDtypeStruct((NPA, H), F32),
        jax.ShapeDtypeStruct((2, NPA, H), F32),
        jax.ShapeDtypeStruct((2, NPT, H), F32),
    ),
    scratch_types=[
        pltpu.VMEM((40, CH), I32),
        pltpu.VMEM((CH, H), F32),
        pltpu.VMEM((80,), I32),
        pltpu.VMEM((80, H), F32),
        pltpu.VMEM_SHARED((NPA, H), F32),
        pltpu.VMEM_SHARED((NPT, H), F32),
        pltpu.SemaphoreType.DMA,
    ],
)
def _sc_init(lab_hbm, emb_tab_hbm, aa_d0, aa_d1, ta_d0, ta_d1, at_d0, at_d1,
             zrows_hbm, ones_hbm,
             emb_out, dega_out, degt_out,
             dbuf, onesv, lidx, erows, dega, degt, sem):
    c = lax.axis_index("c")
    s = lax.axis_index("s")
    w = c * 16 + s
    pltpu.sync_copy(zrows_hbm, dega.at[pl.ds(s * ROWS_A, ROWS_A)])
    pltpu.sync_copy(zrows_hbm.at[pl.ds(0, ROWS_T)], degt.at[pl.ds(s * ROWS_T, ROWS_T)])
    pltpu.sync_copy(ones_hbm, onesv)
    plsc.subcore_barrier()

    def deg_loop(dst_ref, stages, acc):
        for off, ncopy, nproc in stages:
            pltpu.sync_copy(dst_ref.at[s, pl.ds(off, ncopy)],
                            dbuf.at[pl.ds(0, ncopy)])

            def body(i, carry):
                pltpu.sync_copy(onesv, acc.at[dbuf.at[i]], add=True)
                return carry
            lax.fori_loop(0, nproc, body, 0)

    @pl.when(c == 0)
    def _():
        deg_loop(aa_d0, AA_ST0, dega)
        deg_loop(ta_d0, AT_ST0, dega)
        deg_loop(at_d0, AT_ST0, degt)

    @pl.when(c == 1)
    def _():
        deg_loop(aa_d1, AA_ST1, dega)
        deg_loop(ta_d1, AT_ST1, dega)
        deg_loop(at_d1, AT_ST1, degt)

    def gbody(j, carry):
        base = pl.multiple_of(w * 320 + j * 80, 8)
        pltpu.sync_copy(lab_hbm.at[pl.ds(base, 80)], lidx)
        pltpu.async_copy(emb_tab_hbm.at[lidx], erows, sem).wait()
        pltpu.sync_copy(erows, emb_out.at[pl.ds(base, 80)])
        return carry
    lax.fori_loop(0, 4, gbody, 0)

    plsc.subcore_barrier()
    pltpu.sync_copy(dega.at[pl.ds(s * ROWS_A, ROWS_A)],
                    dega_out.at[c, pl.ds(s * ROWS_A, ROWS_A)])
    pltpu.sync_copy(degt.at[pl.ds(s * ROWS_T, ROWS_T)],
                    degt_out.at[c, pl.ds(s * ROWS_T, ROWS_T)])


@functools.partial(
    pl.kernel,
    mesh=_MESH,
    out_type=(
        jax.ShapeDtypeStruct((2, NPA, H), F32),
        jax.ShapeDtypeStruct((2, NPT, H), F32),
    ),
    scratch_types=[
        pltpu.VMEM((40, CH), I32),
        pltpu.VMEM((40, CH), I32),
        pltpu.VMEM((CH, H), F32),
        pltpu.VMEM((CH, H), F32),
        pltpu.VMEM_SHARED((NPA, H), F32),
        pltpu.VMEM_SHARED((NPT, H), F32),
        pltpu.SemaphoreType.DMA,
        pltpu.SemaphoreType.DMA,
    ],
)
def _sc_seg(maa_hbm, mta_hbm, mat_hbm,
            aa_s0, aa_d0, aa_s1, aa_d1, ta_s0, ta_d0, ta_s1, ta_d1,
            at_s0, at_d0, at_s1, at_d1, zrows_hbm,
            outa, outt,
            sbuf, dbuf, rows_a, rows_b, acca, acct, sem_a, sem_b):
    c = lax.axis_index("c")
    s = lax.axis_index("s")
    pltpu.sync_copy(zrows_hbm, acca.at[pl.ds(s * ROWS_A, ROWS_A)])
    pltpu.sync_copy(zrows_hbm.at[pl.ds(0, ROWS_T)], acct.at[pl.ds(s * ROWS_T, ROWS_T)])
    plsc.subcore_barrier()

    def seg_loop(src_ref, dst_ref, tab_ref, acc, stages):
        for off, ncopy, nproc in stages:
            pltpu.sync_copy(src_ref.at[s, pl.ds(off, ncopy)],
                            sbuf.at[pl.ds(0, ncopy)])
            pltpu.sync_copy(dst_ref.at[s, pl.ds(off, ncopy)],
                            dbuf.at[pl.ds(0, ncopy)])
            pltpu.async_copy(tab_ref.at[sbuf.at[0]], rows_a, sem_a)
            npair = nproc // 2

            def body(i, carry):
                a = 2 * i
                b = a + 1
                pltpu.async_copy(tab_ref.at[sbuf.at[b]], rows_b, sem_b)
                pltpu.make_async_copy(tab_ref.at[sbuf.at[a]], rows_a, sem_a).wait()
                pltpu.sync_copy(rows_a, acc.at[dbuf.at[a]], add=True)

                @pl.when(i < npair - 1)
                def _():
                    pltpu.async_copy(tab_ref.at[sbuf.at[a + 2]], rows_a, sem_a)

                pltpu.make_async_copy(tab_ref.at[sbuf.at[b]], rows_b, sem_b).wait()
                pltpu.sync_copy(rows_b, acc.at[dbuf.at[b]], add=True)
                return carry
            lax.fori_loop(0, npair, body, 0)

    @pl.when(c == 0)
    def _():
        seg_loop(aa_s0, aa_d0, maa_hbm, acca, AA_ST0)
        seg_loop(ta_s0, ta_d0, mta_hbm, acca, AT_ST0)
        seg_loop(at_s0, at_d0, mat_hbm, acct, AT_ST0)

    @pl.when(c == 1)
    def _():
        seg_loop(ta_s1, ta_d1, mta_hbm, acca, AT_ST1)
        seg_loop(at_s1, at_d1, mat_hbm, acct, AT_ST1)
        seg_loop(aa_s1, aa_d1, maa_hbm, acca, AA_ST1)

    plsc.subcore_barrier()
    pltpu.sync_copy(acca.at[pl.ds(s * ROWS_A, ROWS_A)],
                    outa.at[c, pl.ds(s * ROWS_A, ROWS_A)])
    pltpu.sync_copy(acct.at[pl.ds(s * ROWS_T, ROWS_T)],
                    outt.at[c, pl.ds(s * ROWS_T, ROWS_T)])


GB = 1000
GRID = N_AST // GB

_blk = lambda shape, imap: pl.BlockSpec(shape, imap)
_whole = lambda shape: pl.BlockSpec(shape, lambda i: tuple(0 for _ in shape))


def _tc_pre(emb, content, Wc, bc, temb, Waa, Wat, Wta):
    def body(emb_ref, cont_ref, Wc_ref, bc_ref, temb_ref, Waa_ref, Wat_ref,
             Wta_ref, maa_ref, mat_ref, mta_ref):
        h0 = jnp.concatenate(
            [emb_ref[:, :64],
             jnp.dot(cont_ref[...], Wc_ref[...], preferred_element_type=F32)
             + bc_ref[...]], axis=1)
        maa_ref[...] = jnp.dot(h0, Waa_ref[...], preferred_element_type=F32)
        mat_ref[...] = jnp.dot(h0, Wat_ref[...], preferred_element_type=F32)

        @pl.when(pl.program_id(0) == 0)
        def _():
            row = jnp.dot(temb_ref[...], Wta_ref[...], preferred_element_type=F32)
            mta_ref[...] = jnp.broadcast_to(row, (N_TEST, H))

    return pl.pallas_call(
        body,
        grid=(GRID,),
        in_specs=[
            _blk((GB, H), lambda i: (i, 0)),
            _blk((GB, H), lambda i: (i, 0)),
            _whole((H, 64)),
            _whole((1, 64)),
            _whole((1, H)),
            _whole((H, H)),
            _whole((H, H)),
            _whole((H, H)),
        ],
        out_specs=[
            _blk((GB, H), lambda i: (i, 0)),
            _blk((GB, H), lambda i: (i, 0)),
            _whole((N_TEST, H)),
        ],
        out_shape=[
            jax.ShapeDtypeStruct((N_AST, H), F32),
            jax.ShapeDtypeStruct((N_AST, H), F32),
            jax.ShapeDtypeStruct((N_TEST, H), F32),
        ],
    )(emb, content, Wc, bc, temb, Waa, Wat, Wta)


def _tc_combine(acca, acct, dega, degt, ba, bt, res, nxt, dec, emit_h):
    with_res = res is not None
    with_nxt = nxt is not None
    final = dec is not None

    def body(*refs):
        it = iter(refs)
        acca_ref = next(it); acct_ref = next(it)
        dega_ref = next(it); degt_ref = next(it)
        ba_ref = next(it); bt_ref = next(it)
        if with_res:
            resa_ref = next(it); rest_ref = next(it)
        if with_nxt:
            Waa_ref = next(it); Wat_ref = next(it); Wta_ref = next(it)
        if final:
            Wd_ref = next(it); bd_ref = next(it)
        if emit_h:
            ha_ref = next(it); ht_ref = next(it)
        if with_nxt:
            maa_ref = next(it); mat_ref = next(it); mta_ref = next(it)
        if final:
            lg_ref = next(it); pr_ref = next(it)

        agg = acca_ref[0] + acca_ref[1]
        deg = jnp.maximum(dega_ref[0, :, 0:1] + dega_ref[1, :, 0:1], 1.0)
        h = jnp.maximum(agg / deg + ba_ref[...], 0.0)
        if with_res:
            h = resa_ref[...] + h
        if emit_h:
            ha_ref[...] = h
        if with_nxt:
            maa_ref[...] = jnp.dot(h, Waa_ref[...], preferred_element_type=F32)
            mat_ref[...] = jnp.dot(h, Wat_ref[...], preferred_element_type=F32)
        if final:
            lg = jnp.dot(h, Wd_ref[...], preferred_element_type=F32) + bd_ref[...]
            lg_ref[...] = lg
            m = jnp.max(lg, axis=1, keepdims=True)
            e = jnp.exp(lg - m)
            pr_ref[...] = e / jnp.sum(e, axis=1, keepdims=True)

        @pl.when(pl.program_id(0) == 0)
        def _():
            agg_t = acct_ref[0, :N_TEST] + acct_ref[1, :N_TEST]
            deg_t = jnp.maximum(degt_ref[0, :N_TEST, 0:1]
                                + degt_ref[1, :N_TEST, 0:1], 1.0)
            ht = jnp.maximum(agg_t / deg_t + bt_ref[...], 0.0)
            if with_res:
                ht = rest_ref[...] + ht
            if emit_h:
                ht_ref[...] = ht
            if with_nxt:
                mta_ref[...] = jnp.dot(ht, Wta_ref[...], preferred_element_type=F32)

    in_specs = [
        _blk((2, GB, H), lambda i: (0, i, 0)),
        _whole((2, NPT, H)),
        _blk((2, GB, H), lambda i: (0, i, 0)),
        _whole((2, NPT, H)),
        _whole((1, H)),
        _whole((1, H)),
    ]
    args = [acca, acct, dega, degt, ba, bt]
    if with_res:
        in_specs += [_blk((GB, H), lambda i: (i, 0)), _whole((N_TEST, H))]
        args += [res[0], res[1]]
    if with_nxt:
        in_specs += [_whole((H, H))] * 3
        args += list(nxt)
    if final:
        in_specs += [_whole((H, 3)), _whole((1, 3))]
        args += list(dec)

    out_specs, out_shape = [], []
    if emit_h:
        out_specs += [_blk((GB, H), lambda i: (i, 0)), _whole((N_TEST, H))]
        out_shape += [jax.ShapeDtypeStruct((N_AST, H), F32),
                      jax.ShapeDtypeStruct((N_TEST, H), F32)]
    if with_nxt:
        out_specs += [_blk((GB, H), lambda i: (i, 0)),
                      _blk((GB, H), lambda i: (i, 0)),
                      _whole((N_TEST, H))]
        out_shape += [jax.ShapeDtypeStruct((N_AST, H), F32),
                      jax.ShapeDtypeStruct((N_AST, H), F32),
                      jax.ShapeDtypeStruct((N_TEST, H), F32)]
    if final:
        out_specs += [_blk((GB, 3), lambda i: (i, 0)),
                      _blk((GB, 3), lambda i: (i, 0))]
        out_shape += [jax.ShapeDtypeStruct((N_AST, 3), F32),
                      jax.ShapeDtypeStruct((N_AST, 3), F32)]

    return pl.pallas_call(
        body, grid=(GRID,), in_specs=in_specs, out_specs=out_specs,
        out_shape=out_shape)(*args)


def _pad_edges(src, dst, e, r0, r1, ws1, trash, trash_n):
    e_pad = 16 * (r0 + r1) * CH
    pad = e_pad - e
    tr = trash + (jnp.arange(pad, dtype=I32) % trash_n)
    s1d = jnp.concatenate([src, jnp.zeros((pad,), I32)])
    d1d = jnp.concatenate([dst, tr])
    cut = 16 * r0 * CH
    out = []
    for a in (s1d, d1d):
        a0 = a[:cut].reshape(16, r0, CH)
        a1 = a[cut:].reshape(16, r1, CH)
        if ws1 != r1:
            a1 = jnp.pad(a1, ((0, 0), (0, ws1 - r1), (0, 0)))
        out += [a0, a1]
    return out


def kernel(ast_label, ast_content, astast_src, astast_dst, asttest_src,
           asttest_dst, testast_src, testast_dst, params):
    aa_s0, aa_s1, aa_d0, aa_d1 = _pad_edges(
        astast_src, astast_dst, E_AA, AA0, AA1, AA1, N_AST, NPA - N_AST)
    at_s0, at_s1, at_d0, at_d1 = _pad_edges(
        asttest_src, asttest_dst, E_AT, AT0, AT1, WS_AT1, N_TEST, NPT - N_TEST)
    ta_s0, ta_s1, ta_d0, ta_d1 = _pad_edges(
        testast_src, testast_dst, E_AT, AT0, AT1, WS_AT1, N_AST, NPA - N_AST)
    lab = jnp.concatenate([ast_label, jnp.zeros((NPA - N_AST,), I32)])

    ones128 = jnp.ones((CH, H), F32)
    zrows = jnp.zeros((ROWS_A, H), F32)
    emb_tab = jnp.pad(params["ast_label_emb"], ((0, 0), (0, H - 64)))

    emb, dega, degt = _sc_init(lab, emb_tab, aa_d0, aa_d1, ta_d0, ta_d1,
                               at_d0, at_d1, zrows, ones128)

    bc = params["ast_content_b"].reshape(1, 64)
    temb = params["test_embedding"].reshape(1, H)
    wl = lambda l: (params["l%d_W_astast" % l], params["l%d_W_asttest" % l],
                    params["l%d_W_testast" % l])
    bl = lambda l: (params["l%d_b_ast" % l].reshape(1, H),
                    params["l%d_b_test" % l].reshape(1, H))

    maa, mat, mta = _tc_pre(emb, ast_content, params["ast_content_W"], bc,
                            temb, *wl(1))

    def seg(maa, mat, mta):
        return _sc_seg(maa, mta, mat, aa_s0, aa_d0, aa_s1, aa_d1,
                       ta_s0, ta_d0, ta_s1, ta_d1,
                       at_s0, at_d0, at_s1, at_d1, zrows)

    acca, acct = seg(maa, mat, mta)
    b1a, b1t = bl(1)
    h1a, h1t, maa, mat, mta = _tc_combine(acca, acct, dega, degt, b1a, b1t,
                                          None, wl(2), None, True)
    acca, acct = seg(maa, mat, mta)
    b2a, b2t = bl(2)
    maa, mat, mta = _tc_combine(acca, acct, dega, degt, b2a, b2t,
                                (h1a, h1t), wl(3), None, False)
    acca, acct = seg(maa, mat, mta)
    b3a, b3t = bl(3)
    h3a, h3t, maa, mat, mta = _tc_combine(acca, acct, dega, degt, b3a, b3t,
                                          None, wl(4), None, True)
    acca, acct = seg(maa, mat, mta)
    b4a, b4t = bl(4)
    maa, mat, mta = _tc_combine(acca, acct, dega, degt, b4a, b4t,
                                (h3a, h3t), wl(5), None, False)
    acca, acct = seg(maa, mat, mta)
    b5a, b5t = bl(5)
    dec = (params["ast_dec_W"], params["ast_dec_b"].reshape(1, 3))
    h5a, h5t, logits, pred = _tc_combine(acca, acct, dega, degt, b5a, b5t,
                                         None, None, dec, True)
    return h5a, h5t, logits, pred
